# Initial kernel scaffold; baseline (speedup 1.0000x reference)
import jax
import jax.numpy as jnp
from jax.experimental import pallas as pl


def kernel(card_table, relic_table, potion_table, monster_table, node_type_table, action_type_table, text_token_table, rest_option_table, event_option_table, index_table, card_idx, relic_idx, potion_idx, monster_idx, node_type_idx, action_type_idx, text_token_idx, rest_option_idx, event_option_idx, index_idx):
    raise NotImplementedError("write your pallas kernel here")


# speedup vs baseline: 2.6748x; 2.6748x over previous
import jax
import jax.numpy as jnp
from jax import lax
from jax.experimental import pallas as pl
from jax.experimental.pallas import tpu as pltpu
from jax.experimental.pallas import tpu_sc as plsc

EMBED_DIM = 32
NC, NS = 2, 16
NW = NC * NS
CHUNK = 128

_TABLES = (
    ("card", 100000, 4096 * 50),
    ("relic", 100000, 4096 * 50),
    ("potion", 100000, 4096 * 5),
    ("monster", 100000, 4096 * 8),
    ("node", 8, 4096),
    ("action", 22, 4096),
    ("text", 2048, 4096 * 50),
    ("rest", 8, 4096),
    ("event", 5, 4096),
    ("index", 20, 4096 * 20),
)
_NCHUNKS = tuple(rows // (NW * CHUNK) for (_, _, rows) in _TABLES)
_KBUF = {50: 5, 20: 5, 8: 4, 5: 5, 1: 1}
MAXK = 5


def _body(*refs):
    tables = refs[0:10]
    idxs = refs[10:20]
    outs = refs[20:30]
    idx_bufs = refs[30:40]
    rows = refs[40:40 + MAXK]
    gsems = refs[45:45 + MAXK]
    ssems = refs[50:50 + MAXK]

    wid = lax.axis_index("s") * NC + lax.axis_index("c")

    for t in range(10):
        n = _NCHUNKS[t]
        pltpu.sync_copy(idxs[t].at[pl.ds(wid * n * CHUNK, n * CHUNK)],
                        idx_bufs[t])

    for t in range(10):
        n = _NCHUNKS[t]
        k = _KBUF[n]
        tbl, idxb, out = tables[t], idx_bufs[t], outs[t]
        base_row = wid * n * CHUNK

        def fire_gather(c, b):
            pltpu.async_copy(tbl.at[idxb.at[pl.ds(c * CHUNK, CHUNK)]],
                             rows[b], gsems[b])

        def wait_gather(c, b):
            pltpu.make_async_copy(tbl.at[idxb.at[pl.ds(c * CHUNK, CHUNK)]],
                                  rows[b], gsems[b]).wait()

        def fire_store(c, b):
            pltpu.async_copy(rows[b], out.at[pl.ds(base_row + c * CHUNK, CHUNK)],
                             ssems[b])

        def wait_store(c, b):
            pltpu.make_async_copy(rows[b],
                                  out.at[pl.ds(base_row + c * CHUNK, CHUNK)],
                                  ssems[b]).wait()

        for b in range(k):
            fire_gather(b, b)

        g_total = n // k
        if g_total > 1:
            def group(g, carry):
                for b in range(k):
                    c = g * k + b
                    wait_gather(c, b)
                    fire_store(c, b)
                for b in range(k):
                    c = g * k + b
                    wait_store(c, b)
                    fire_gather(c + k, b)
                return carry

            lax.fori_loop(0, g_total - 1, group, 0)

        for b in range(k):
            c = (g_total - 1) * k + b
            wait_gather(c, b)
            fire_store(c, b)
        for b in range(k):
            c = (g_total - 1) * k + b
            wait_store(c, b)


def _build():
    out_type = tuple(
        jax.ShapeDtypeStruct((rows, EMBED_DIM), jnp.float32)
        for (_, _, rows) in _TABLES
    )
    scratch = (
        [pltpu.VMEM((n * CHUNK,), jnp.int32) for n in _NCHUNKS]
        + [pltpu.VMEM((CHUNK, EMBED_DIM), jnp.float32) for _ in range(MAXK)]
        + [pltpu.SemaphoreType.DMA for _ in range(2 * MAXK)]
    )
    mesh = plsc.VectorSubcoreMesh(core_axis_name="c", subcore_axis_name="s")
    return pl.kernel(
        _body, out_type=out_type, mesh=mesh, scratch_types=scratch,
        compiler_params=pltpu.CompilerParams(use_tc_tiling_on_sc=False))


def kernel(card_table, relic_table, potion_table, monster_table,
           node_type_table, action_type_table, text_token_table,
           rest_option_table, event_option_table, index_table,
           card_idx, relic_idx, potion_idx, monster_idx,
           node_type_idx, action_type_idx, text_token_idx,
           rest_option_idx, event_option_idx, index_idx):
    tables = (card_table, relic_table, potion_table, monster_table,
              node_type_table, action_type_table, text_token_table,
              rest_option_table, event_option_table, index_table)
    raw_idx = (card_idx, relic_idx, potion_idx, monster_idx,
               node_type_idx, action_type_idx, text_token_idx,
               rest_option_idx, event_option_idx, index_idx)
    flat_idx = tuple(ix.reshape(-1) for ix in raw_idx)

    outs = _build()(*tables, *flat_idx)

    return tuple(
        o.reshape(ix.shape + (EMBED_DIM,))
        for o, ix in zip(outs, raw_idx)
    )


# speedup vs baseline: 3.3491x; 1.2521x over previous
import jax
import jax.numpy as jnp
from jax import lax
from jax.experimental import pallas as pl
from jax.experimental.pallas import tpu as pltpu
from jax.experimental.pallas import tpu_sc as plsc

EMBED_DIM = 32
NC, NS = 2, 16
NW = NC * NS
CHUNK = 128
B = 4096

_SLOTS = (50, 50, 5, 8, 1, 1, 50, 1, 1, 20)
_KBUF = {50: 5, 20: 5, 8: 4, 5: 5, 1: 1}
MAXK = 5


def _transpose_chunk(rows, tbuf):
    base = lax.iota(jnp.int32, 16)

    def col_body(c, carry):
        cols = jnp.full((16,), c, jnp.int32)
        for g in range(8):
            v = plsc.load_gather(rows, [base + (g * 16), cols])
            tbuf[c, pl.ds(g * 16, 16)] = v
        return carry

    lax.fori_loop(0, EMBED_DIM, col_body, 0)


def _body(*refs):
    tables = refs[0:10]
    idxs = refs[10:20]
    outs = refs[20:30]
    idx_bufs = refs[30:40]
    rows = refs[40:40 + MAXK]
    tbufs = refs[45:45 + MAXK]
    gsems = refs[50:50 + MAXK]
    ssems = refs[55:55 + MAXK]

    wid = lax.axis_index("s") * NC + lax.axis_index("c")

    for t in range(10):
        pltpu.sync_copy(idxs[t].at[:, pl.ds(wid * CHUNK, CHUNK)], idx_bufs[t])

    for t in range(10):
        n = _SLOTS[t]
        k = _KBUF[n]
        tbl, idxb, out = tables[t], idx_bufs[t], outs[t]

        def fire_gather(c, b):
            pltpu.async_copy(tbl.at[idxb.at[c]], rows[b], gsems[b])

        def wait_gather(c, b):
            pltpu.make_async_copy(tbl.at[idxb.at[c]], rows[b],
                                  gsems[b]).wait()

        def dst(c, cr):
            return out.at[c, cr, wid]

        def fire_stores(c, b):
            for cr in range(4):
                pltpu.async_copy(tbufs[b].at[pl.ds(cr * 8, 8)], dst(c, cr),
                                 ssems[b])

        def wait_stores(c, b):
            for cr in range(4):
                pltpu.make_async_copy(tbufs[b].at[pl.ds(cr * 8, 8)],
                                      dst(c, cr), ssems[b]).wait()

        for bb in range(k):
            fire_gather(bb, bb)

        g_total = n // k
        if g_total > 1:
            def group(g, carry):
                for bb in range(k):
                    c = g * k + bb
                    wait_gather(c, bb)
                    _transpose_chunk(rows[bb], tbufs[bb])
                    fire_stores(c, bb)
                for bb in range(k):
                    c = g * k + bb
                    wait_stores(c, bb)
                    fire_gather(c + k, bb)
                return carry

            lax.fori_loop(0, g_total - 1, group, 0)

        for bb in range(k):
            c = (g_total - 1) * k + bb
            wait_gather(c, bb)
            _transpose_chunk(rows[bb], tbufs[bb])
            fire_stores(c, bb)
        for bb in range(k):
            c = (g_total - 1) * k + bb
            wait_stores(c, bb)


def _build():
    out_type = tuple(
        jax.ShapeDtypeStruct((n, 4, EMBED_DIM, 8, CHUNK), jnp.float32)
        for n in _SLOTS
    )
    scratch = (
        [pltpu.VMEM((n, CHUNK), jnp.int32) for n in _SLOTS]
        + [pltpu.VMEM((CHUNK, EMBED_DIM), jnp.float32) for _ in range(MAXK)]
        + [pltpu.VMEM((EMBED_DIM, CHUNK), jnp.float32) for _ in range(MAXK)]
        + [pltpu.SemaphoreType.DMA for _ in range(2 * MAXK)]
    )
    mesh = plsc.VectorSubcoreMesh(core_axis_name="c", subcore_axis_name="s")
    return pl.kernel(
        _body, out_type=out_type, mesh=mesh, scratch_types=scratch,
        compiler_params=pltpu.CompilerParams(use_tc_tiling_on_sc=False,
                                             needs_layout_passes=False))


def kernel(card_table, relic_table, potion_table, monster_table,
           node_type_table, action_type_table, text_token_table,
           rest_option_table, event_option_table, index_table,
           card_idx, relic_idx, potion_idx, monster_idx,
           node_type_idx, action_type_idx, text_token_idx,
           rest_option_idx, event_option_idx, index_idx):
    tables = (card_table, relic_table, potion_table, monster_table,
              node_type_table, action_type_table, text_token_table,
              rest_option_table, event_option_table, index_table)
    raw_idx = (card_idx, relic_idx, potion_idx, monster_idx,
               node_type_idx, action_type_idx, text_token_idx,
               rest_option_idx, event_option_idx, index_idx)
    idx_t = tuple(
        ix.reshape(B, -1).T for ix in raw_idx
    )

    outs = _build()(*tables, *idx_t)

    result = []
    for o, ix in zip(outs, raw_idx):
        r = o.transpose(2, 4, 0, 1, 3).reshape(B, o.shape[0], EMBED_DIM)
        if ix.ndim == 1:
            r = r.reshape(B, EMBED_DIM)
        result.append(r)
    return tuple(result)


# speedup vs baseline: 3.8882x; 1.1610x over previous
import jax
import jax.numpy as jnp
from jax import lax
from jax.experimental import pallas as pl
from jax.experimental.pallas import tpu as pltpu
from jax.experimental.pallas import tpu_sc as plsc

EMBED_DIM = 32
NC, NS = 2, 16
NW = NC * NS
CHUNK = 128
B = 4096
K = 8

_SLOTS = (50, 50, 5, 8, 1, 1, 50, 1, 1, 20)


def _body(*refs):
    tables = refs[0:10]
    idxs = refs[10:20]
    outs = refs[20:30]
    idx_bufs = refs[30:40]
    rows = refs[40]
    tbuf = refs[41]
    gsems = refs[42]
    ssems = refs[43]

    wid = lax.axis_index("s") * NC + lax.axis_index("c")
    lane = lax.iota(jnp.int32, 16)

    for t in range(10):
        pltpu.sync_copy(idxs[t].at[:, pl.ds(wid * CHUNK, CHUNK)], idx_bufs[t])

    for t in range(10):
        n = _SLOTS[t]
        tbl, idxb, out = tables[t], idx_bufs[t], outs[t]

        def fire_gather(c, rb):
            pltpu.async_copy(tbl.at[idxb.at[c]],
                             rows.at[pl.ds(rb * CHUNK, CHUNK)], gsems.at[rb])

        def wait_gather(c, rb):
            pltpu.make_async_copy(tbl.at[idxb.at[c]],
                                  rows.at[pl.ds(rb * CHUNK, CHUNK)],
                                  gsems.at[rb]).wait()

        def fire_stores(c, rb):
            for cr in range(4):
                pltpu.async_copy(tbuf.at[pl.ds(rb * EMBED_DIM + cr * 8, 8)],
                                 out.at[c, cr, wid], ssems.at[rb])

        def wait_stores(c, rb):
            for cr in range(4):
                pltpu.make_async_copy(
                    tbuf.at[pl.ds(rb * EMBED_DIM + cr * 8, 8)],
                    out.at[c, cr, wid], ssems.at[rb]).wait()

        def transpose(rb):
            rbase = rb * CHUNK
            tbase = rb * EMBED_DIM
            for c in range(EMBED_DIM):
                cols = jnp.full((16,), c, jnp.int32)
                vs = [plsc.load_gather(rows, [rbase + lane + g * 16, cols])
                      for g in range(8)]
                for g in range(8):
                    tbuf[tbase + c, pl.ds(g * 16, 16)] = vs[g]

        k = min(K, n)
        for c0 in range(k):
            fire_gather(c0, c0)

        def chunk_body(c, carry):
            rb = lax.rem(c, k)
            wait_gather(c, rb)

            @pl.when(c >= k)
            def _():
                wait_stores(c - k, rb)

            transpose(rb)
            fire_stores(c, rb)

            @pl.when(c + k < n)
            def _():
                fire_gather(c + k, rb)
            return carry

        lax.fori_loop(0, n, chunk_body, 0)

        for i in range(k):
            wait_stores(n - k + i, (n - k + i) % k)


def _build():
    out_type = tuple(
        jax.ShapeDtypeStruct((n, 4, EMBED_DIM, 8, CHUNK), jnp.float32)
        for n in _SLOTS
    )
    scratch = (
        [pltpu.VMEM((n, CHUNK), jnp.int32) for n in _SLOTS]
        + [pltpu.VMEM((K * CHUNK, EMBED_DIM), jnp.float32),
           pltpu.VMEM((K * EMBED_DIM, CHUNK), jnp.float32),
           pltpu.SemaphoreType.DMA((K,)),
           pltpu.SemaphoreType.DMA((K,))]
    )
    mesh = plsc.VectorSubcoreMesh(core_axis_name="c", subcore_axis_name="s")
    return pl.kernel(
        _body, out_type=out_type, mesh=mesh, scratch_types=scratch,
        compiler_params=pltpu.CompilerParams(use_tc_tiling_on_sc=False,
                                             needs_layout_passes=False))


def kernel(card_table, relic_table, potion_table, monster_table,
           node_type_table, action_type_table, text_token_table,
           rest_option_table, event_option_table, index_table,
           card_idx, relic_idx, potion_idx, monster_idx,
           node_type_idx, action_type_idx, text_token_idx,
           rest_option_idx, event_option_idx, index_idx):
    tables = (card_table, relic_table, potion_table, monster_table,
              node_type_table, action_type_table, text_token_table,
              rest_option_table, event_option_table, index_table)
    raw_idx = (card_idx, relic_idx, potion_idx, monster_idx,
               node_type_idx, action_type_idx, text_token_idx,
               rest_option_idx, event_option_idx, index_idx)
    idx_t = tuple(
        ix.reshape(B, -1).T for ix in raw_idx
    )

    outs = _build()(*tables, *idx_t)

    result = []
    for o, ix in zip(outs, raw_idx):
        r = o.transpose(2, 4, 0, 1, 3).reshape(B, o.shape[0], EMBED_DIM)
        if ix.ndim == 1:
            r = r.reshape(B, EMBED_DIM)
        result.append(r)
    return tuple(result)


# speedup vs baseline: 4.1645x; 1.0711x over previous
import jax
import jax.numpy as jnp
from jax import lax
from jax.experimental import pallas as pl
from jax.experimental.pallas import tpu as pltpu
from jax.experimental.pallas import tpu_sc as plsc

EMBED_DIM = 32
NC, NS = 2, 16
NW = NC * NS
CHUNK = 128
B = 4096
K = 8

_SLOTS = (50, 50, 5, 8, 1, 1, 50, 1, 1, 20)
_GROUP_A = (0, 1)
_GROUP_B = (2, 3, 4, 5, 6, 7, 8, 9)


def _make_body(tids):
    nt = len(tids)

    def _body(*refs):
        tables = refs[0:nt]
        idxs = refs[nt:2 * nt]
        outs = refs[2 * nt:3 * nt]
        idx_bufs = refs[3 * nt:4 * nt]
        rows = refs[4 * nt]
        tbuf = refs[4 * nt + 1]
        gsems = refs[4 * nt + 2]
        ssems = refs[4 * nt + 3]

        wid = lax.axis_index("s") * NC + lax.axis_index("c")
        lane = lax.iota(jnp.int32, 16)

        for t in range(nt):
            pltpu.sync_copy(idxs[t].at[:, pl.ds(wid * CHUNK, CHUNK)],
                            idx_bufs[t])

        for t in range(nt):
            n = _SLOTS[tids[t]]
            tbl, idxb, out = tables[t], idx_bufs[t], outs[t]

            def fire_gather(c, rb):
                pltpu.async_copy(tbl.at[idxb.at[c]],
                                 rows.at[pl.ds(rb * CHUNK, CHUNK)],
                                 gsems.at[rb])

            def wait_gather(c, rb):
                pltpu.make_async_copy(tbl.at[idxb.at[c]],
                                      rows.at[pl.ds(rb * CHUNK, CHUNK)],
                                      gsems.at[rb]).wait()

            def fire_stores(c, rb):
                for cr in range(4):
                    pltpu.async_copy(
                        tbuf.at[pl.ds(rb * EMBED_DIM + cr * 8, 8)],
                        out.at[c, cr, wid], ssems.at[rb])

            def wait_stores(c, rb):
                for cr in range(4):
                    pltpu.make_async_copy(
                        tbuf.at[pl.ds(rb * EMBED_DIM + cr * 8, 8)],
                        out.at[c, cr, wid], ssems.at[rb]).wait()

            def transpose(rb):
                rbase = rb * CHUNK
                tbase = rb * EMBED_DIM
                for c in range(EMBED_DIM):
                    cols = jnp.full((16,), c, jnp.int32)
                    vs = [plsc.load_gather(rows,
                                           [rbase + lane + g * 16, cols])
                          for g in range(8)]
                    for g in range(8):
                        tbuf[tbase + c, pl.ds(g * 16, 16)] = vs[g]

            k = min(K, n)
            for c0 in range(k):
                fire_gather(c0, c0)

            def chunk_body(c, carry):
                rb = lax.rem(c, k)
                wait_gather(c, rb)

                @pl.when(c >= k)
                def _():
                    wait_stores(c - k, rb)

                transpose(rb)
                fire_stores(c, rb)

                @pl.when(c + k < n)
                def _():
                    fire_gather(c + k, rb)
                return carry

            lax.fori_loop(0, n, chunk_body, 0)

            for i in range(k):
                wait_stores(n - k + i, (n - k + i) % k)

    return _body


def _build(tids):
    out_type = tuple(
        jax.ShapeDtypeStruct((_SLOTS[t], 4, EMBED_DIM, 8, CHUNK),
                             jnp.float32)
        for t in tids
    )
    scratch = (
        [pltpu.VMEM((_SLOTS[t], CHUNK), jnp.int32) for t in tids]
        + [pltpu.VMEM((K * CHUNK, EMBED_DIM), jnp.float32),
           pltpu.VMEM((K * EMBED_DIM, CHUNK), jnp.float32),
           pltpu.SemaphoreType.DMA((K,)),
           pltpu.SemaphoreType.DMA((K,))]
    )
    mesh = plsc.VectorSubcoreMesh(core_axis_name="c", subcore_axis_name="s")
    return pl.kernel(
        _make_body(tids), out_type=out_type, mesh=mesh,
        scratch_types=scratch,
        compiler_params=pltpu.CompilerParams(use_tc_tiling_on_sc=False,
                                             needs_layout_passes=False))


def kernel(card_table, relic_table, potion_table, monster_table,
           node_type_table, action_type_table, text_token_table,
           rest_option_table, event_option_table, index_table,
           card_idx, relic_idx, potion_idx, monster_idx,
           node_type_idx, action_type_idx, text_token_idx,
           rest_option_idx, event_option_idx, index_idx):
    tables = (card_table, relic_table, potion_table, monster_table,
              node_type_table, action_type_table, text_token_table,
              rest_option_table, event_option_table, index_table)
    raw_idx = (card_idx, relic_idx, potion_idx, monster_idx,
               node_type_idx, action_type_idx, text_token_idx,
               rest_option_idx, event_option_idx, index_idx)
    idx_t = tuple(ix.reshape(B, -1).T for ix in raw_idx)

    outs_a = _build(_GROUP_A)(*[tables[t] for t in _GROUP_A],
                              *[idx_t[t] for t in _GROUP_A])
    outs_b = _build(_GROUP_B)(*[tables[t] for t in _GROUP_B],
                              *[idx_t[t] for t in _GROUP_B])
    outs = list(outs_a) + list(outs_b)
    order = list(_GROUP_A) + list(_GROUP_B)
    outs = [outs[order.index(i)] for i in range(10)]

    result = []
    for o, ix in zip(outs, raw_idx):
        r = o.transpose(2, 4, 0, 1, 3).reshape(B, o.shape[0], EMBED_DIM)
        if ix.ndim == 1:
            r = r.reshape(B, EMBED_DIM)
        result.append(r)
    return tuple(result)
